# baseline (speedup 1.0000x reference)
import functools

import jax
import jax.numpy as jnp
from jax import lax
from jax.experimental import pallas as pl
from jax.experimental.pallas import tpu as pltpu
from jax.experimental.pallas import tpu_sc as plsc

VOCAB = 100000
HIDDEN = 128
NUM_IDS = 4096 * 50

_INFO = plsc.get_sparse_core_info()
NC = _INFO.num_cores
NS = _INFO.num_subcores
NW = NC * NS
B_PER_W = NUM_IDS // NW
CHUNK = 100
N_CHUNKS = B_PER_W // CHUNK
SETB = 4
R = N_CHUNKS // SETB


def _round(table_hbm, out_hbm, idx_v, rows_v, gsem, ssem, wid, r, s, first):
    if not first:
        pltpu.make_async_copy(
            rows_v.at[s], out_hbm.at[wid, pl.ds(0, SETB)], ssem
        ).wait()
    base = r * SETB
    cps = [
        pltpu.async_copy(table_hbm.at[idx_v.at[base + b]], rows_v.at[s, b], gsem)
        for b in range(SETB)
    ]
    for cp in cps:
        cp.wait()
    pltpu.async_copy(rows_v.at[s], out_hbm.at[wid, pl.ds(base, SETB)], ssem)


def _sc_body(ids_hbm, table_hbm, out_hbm, idx_v, rows_v, gsem, ssem):
    wid = lax.axis_index("s") * NC + lax.axis_index("c")
    pltpu.sync_copy(ids_hbm.at[wid], idx_v)

    step = functools.partial(
        _round, table_hbm, out_hbm, idx_v, rows_v, gsem, ssem, wid
    )
    step(0, 0, True)
    step(1, 1, True)

    def body(i, carry):
        step(2 * i, 0, False)
        step(2 * i + 1, 1, False)
        return carry

    lax.fori_loop(1, R // 2, body, 0)

    for s in range(2):
        pltpu.make_async_copy(
            rows_v.at[s], out_hbm.at[wid, pl.ds(0, SETB)], ssem
        ).wait()


@jax.jit
def _emb(ids, table):
    mesh = plsc.VectorSubcoreMesh(core_axis_name="c", subcore_axis_name="s")
    k = functools.partial(
        pl.kernel,
        mesh=mesh,
        out_type=jax.ShapeDtypeStruct((NW, N_CHUNKS, CHUNK, HIDDEN), jnp.float32),
        scratch_types=[
            pltpu.VMEM((N_CHUNKS, CHUNK), jnp.int32),
            pltpu.VMEM((2, SETB, CHUNK, HIDDEN), jnp.float32),
            pltpu.SemaphoreType.DMA,
            pltpu.SemaphoreType.DMA,
        ],
    )(_sc_body)
    return k(ids, table)


def kernel(input_ids, table):
    ids = input_ids.reshape(NW, N_CHUNKS, CHUNK).astype(jnp.int32)
    out = _emb(ids, table)
    return out.reshape(*input_ids.shape, HIDDEN)


# speedup vs baseline: 1.0161x; 1.0161x over previous
import functools

import jax
import jax.numpy as jnp
from jax import lax
from jax.experimental import pallas as pl
from jax.experimental.pallas import tpu as pltpu
from jax.experimental.pallas import tpu_sc as plsc

VOCAB = 100000
HIDDEN = 128
NUM_IDS = 4096 * 50

_INFO = plsc.get_sparse_core_info()
NC = _INFO.num_cores
NS = _INFO.num_subcores
NW = NC * NS
B_PER_W = NUM_IDS // NW
CHUNK = 128
N_CHUNKS = B_PER_W // CHUNK
NBUF = 5


def _sc_body(ids_hbm, table_hbm, out_hbm, idx_v, rows_v, gsem, ssem):
    wid = lax.axis_index("s") * NC + lax.axis_index("c")
    base = wid * B_PER_W
    pltpu.sync_copy(ids_hbm.at[pl.ds(base, B_PER_W)], idx_v)

    def outer(i, carry):
        g = i * NBUF
        copies = []
        for b in range(NBUF):
            idx = idx_v.at[pl.ds((g + b) * CHUNK, CHUNK)]
            copies.append(
                pltpu.async_copy(table_hbm.at[idx], rows_v.at[b], gsem)
            )
        for b in range(NBUF):
            row0 = base + (g + b) * CHUNK
            copies[b].wait()
            pltpu.async_copy(rows_v.at[b], out_hbm.at[pl.ds(row0, CHUNK)], ssem)
        for b in range(NBUF):
            row0 = base + (g + b) * CHUNK
            pltpu.make_async_copy(
                rows_v.at[b], out_hbm.at[pl.ds(row0, CHUNK)], ssem
            ).wait()
        return carry

    lax.fori_loop(0, N_CHUNKS // NBUF, outer, 0)


@jax.jit
def _emb(ids, table):
    mesh = plsc.VectorSubcoreMesh(core_axis_name="c", subcore_axis_name="s")
    k = functools.partial(
        pl.kernel,
        mesh=mesh,
        out_type=jax.ShapeDtypeStruct((NUM_IDS, HIDDEN), jnp.float32),
        scratch_types=[
            pltpu.VMEM((B_PER_W,), jnp.int32),
            pltpu.VMEM((NBUF, CHUNK, HIDDEN), jnp.float32),
            pltpu.SemaphoreType.DMA,
            pltpu.SemaphoreType.DMA,
        ],
    )(_sc_body)
    return k(ids, table)


def kernel(input_ids, table):
    ids = input_ids.reshape(-1).astype(jnp.int32)
    out = _emb(ids, table)
    return out.reshape(*input_ids.shape, HIDDEN)


# speedup vs baseline: 3.0815x; 3.0328x over previous
import functools

import jax
import jax.numpy as jnp
from jax import lax
from jax.experimental import pallas as pl
from jax.experimental.pallas import tpu as pltpu
from jax.experimental.pallas import tpu_sc as plsc

VOCAB = 100000
HIDDEN = 128
NUM_IDS = 4096 * 50

_INFO = plsc.get_sparse_core_info()
NC = _INFO.num_cores
NS = _INFO.num_subcores
NW = NC * NS
B_PER_W = NUM_IDS // NW
CHUNK = 128
N_CHUNKS = B_PER_W // CHUNK
NBUF = 5
R = N_CHUNKS // NBUF


def _sc_body(ids_hbm, table_hbm, out_hbm, idx_v, rows_v, gsem, ssem):
    wid = lax.axis_index("s") * NC + lax.axis_index("c")
    base = wid * B_PER_W
    pltpu.sync_copy(ids_hbm.at[pl.ds(base, B_PER_W)], idx_v)

    def fire_gather(j, b):
        idx = idx_v.at[pl.ds(j * CHUNK, CHUNK)]
        pltpu.async_copy(table_hbm.at[idx], rows_v.at[b], gsem.at[b])

    def wait_gather(b):
        pltpu.make_async_copy(
            table_hbm.at[pl.ds(0, CHUNK)], rows_v.at[b], gsem.at[b]
        ).wait()

    def fire_scatter(j, b):
        pltpu.async_copy(
            rows_v.at[b], out_hbm.at[pl.ds(base + j * CHUNK, CHUNK)], ssem.at[b]
        )

    def wait_scatter(b):
        pltpu.make_async_copy(
            rows_v.at[b], out_hbm.at[pl.ds(base, CHUNK)], ssem.at[b]
        ).wait()

    for b in range(NBUF):
        fire_gather(b, b)

    def body(i, carry):
        g = i * NBUF
        for b in range(NBUF):
            wait_gather(b)
            fire_scatter(g - NBUF + b, b)
        for b in range(NBUF):
            wait_scatter(b)
            fire_gather(g + b, b)
        return carry

    lax.fori_loop(1, R, body, 0)

    for b in range(NBUF):
        wait_gather(b)
        fire_scatter((R - 1) * NBUF + b, b)
    for b in range(NBUF):
        wait_scatter(b)


@jax.jit
def _emb(ids, table):
    mesh = plsc.VectorSubcoreMesh(core_axis_name="c", subcore_axis_name="s")
    k = functools.partial(
        pl.kernel,
        mesh=mesh,
        out_type=jax.ShapeDtypeStruct((NUM_IDS, HIDDEN), jnp.float32),
        scratch_types=[
            pltpu.VMEM((B_PER_W,), jnp.int32),
            pltpu.VMEM((NBUF, CHUNK, HIDDEN), jnp.float32),
            pltpu.SemaphoreType.DMA((NBUF,)),
            pltpu.SemaphoreType.DMA((NBUF,)),
        ],
    )(_sc_body)
    return k(ids, table)


def kernel(input_ids, table):
    batch, seq = input_ids.shape
    ids_t = input_ids.T.reshape(-1).astype(jnp.int32)
    out_t = _emb(ids_t, table)
    return out_t.reshape(seq, batch, HIDDEN).transpose(1, 0, 2)
